# baseline (speedup 1.0000x reference)
import functools

import jax
import jax.numpy as jnp
from jax.experimental import pallas as pl
from jax.experimental.pallas import tpu as pltpu

N = 10000
NP = 10240
E = 320000
HID = 256
ED = 16
NUM_AT_PAD = 104


def _embed_body(an_ref, emb_ref, x0_ref):
    an = an_ref[...]
    ids = jax.lax.broadcasted_iota(jnp.int32, (1, NUM_AT_PAD), 1)
    oh = (an == ids).astype(jnp.float32)
    x0_ref[...] = jnp.dot(oh, emb_ref[...], preferred_element_type=jnp.float32)


def _embed(an2d, emb_pad):
    B = 1024
    return pl.pallas_call(
        _embed_body,
        grid=(NP // B,),
        in_specs=[
            pl.BlockSpec((B, 1), lambda i: (i, 0)),
            pl.BlockSpec((NUM_AT_PAD, 64), lambda i: (0, 0)),
        ],
        out_specs=pl.BlockSpec((B, 64), lambda i: (i, 0)),
        out_shape=jax.ShapeDtypeStruct((NP, 64), jnp.float32),
    )(an2d, emb_pad)


def _edge_mlp_body(ea_ref, w_ref, *out_refs):
    e = jnp.dot(ea_ref[...], w_ref[...], preferred_element_type=jnp.float32)
    offs = [0, 64, 128, 256, 384, 512, 640, 768, 896]
    for r, (lo, hi) in zip(out_refs, zip(offs[:-1], offs[1:])):
        r[...] = e[:, lo:hi]


def _edge_mlp(edge_attr, wcat):
    B = 2000
    widths = [64, 64, 128, 128, 128, 128, 128, 128]
    return pl.pallas_call(
        _edge_mlp_body,
        grid=(E // B,),
        in_specs=[
            pl.BlockSpec((B, ED), lambda i: (i, 0)),
            pl.BlockSpec((ED, 896), lambda i: (0, 0)),
        ],
        out_specs=[pl.BlockSpec((B, w), lambda i: (i, 0)) for w in widths],
        out_shape=[jax.ShapeDtypeStruct((E, w), jnp.float32) for w in widths],
    )(edge_attr, wcat)


def _mlp_body(x0_ref, x1_ref, a0_ref, a1_ref, w1_ref, b1_ref, w2_ref, b2_ref,
              h2_ref, st_ref, *, B):
    h = jnp.concatenate(
        [x0_ref[...] + a0_ref[...], x1_ref[...] + a1_ref[...]], axis=1)
    h1 = jnp.maximum(
        jnp.dot(h, w1_ref[...], preferred_element_type=jnp.float32) + b1_ref[...], 0.0)
    h2 = jnp.dot(h1, w2_ref[...], preferred_element_type=jnp.float32) + b2_ref[...]
    h2_ref[...] = h2
    gid = pl.program_id(0)
    rows = gid * B + jax.lax.broadcasted_iota(jnp.int32, (B, 1), 0)
    hm = h2 * (rows < N).astype(jnp.float32)

    @pl.when(gid == 0)
    def _():
        st_ref[...] = jnp.zeros_like(st_ref)

    st_ref[0:1, :] += jnp.sum(hm, axis=0, keepdims=True)
    st_ref[1:2, :] += jnp.sum(hm * hm, axis=0, keepdims=True)


def _node_mlp(x0, x1, a0, a1, w1p, b1, w2, b2, d):
    B = 512
    return pl.pallas_call(
        functools.partial(_mlp_body, B=B),
        grid=(NP // B,),
        in_specs=[
            pl.BlockSpec((B, d), lambda i: (i, 0)),
            pl.BlockSpec((B, d), lambda i: (i, 0)),
            pl.BlockSpec((B, d), lambda i: (i, 0)),
            pl.BlockSpec((B, d), lambda i: (i, 0)),
            pl.BlockSpec((2 * d, HID), lambda i: (0, 0)),
            pl.BlockSpec((1, HID), lambda i: (0, 0)),
            pl.BlockSpec((HID, HID), lambda i: (0, 0)),
            pl.BlockSpec((1, HID), lambda i: (0, 0)),
        ],
        out_specs=[
            pl.BlockSpec((B, HID), lambda i: (i, 0)),
            pl.BlockSpec((8, HID), lambda i: (0, 0)),
        ],
        out_shape=[
            jax.ShapeDtypeStruct((NP, HID), jnp.float32),
            jax.ShapeDtypeStruct((8, HID), jnp.float32),
        ],
    )(x0, x1, a0, a1, w1p, b1, w2, b2)


def _bn_body(h2_ref, st_ref, g_ref, bt_ref, y0_ref, y1_ref, pool_ref, *, B):
    inv_n = 1.0 / N
    mean = st_ref[0:1, :] * inv_n
    var = st_ref[1:2, :] * inv_n - mean * mean
    inv = jax.lax.rsqrt(var + 1e-5)
    y = jnp.maximum((h2_ref[...] - mean) * inv * g_ref[...] + bt_ref[...], 0.0)
    y0_ref[...] = y[:, :128]
    y1_ref[...] = y[:, 128:]
    gid = pl.program_id(0)
    rows = gid * B + jax.lax.broadcasted_iota(jnp.int32, (B, 1), 0)
    ym = y * (rows < N).astype(jnp.float32)

    @pl.when(gid == 0)
    def _():
        pool_ref[...] = jnp.zeros_like(pool_ref)

    pool_ref[0:1, :] += jnp.sum(ym, axis=0, keepdims=True)


def _bn_apply(h2, st, gamma, beta):
    B = 512
    return pl.pallas_call(
        functools.partial(_bn_body, B=B),
        grid=(NP // B,),
        in_specs=[
            pl.BlockSpec((B, HID), lambda i: (i, 0)),
            pl.BlockSpec((8, HID), lambda i: (0, 0)),
            pl.BlockSpec((1, HID), lambda i: (0, 0)),
            pl.BlockSpec((1, HID), lambda i: (0, 0)),
        ],
        out_specs=[
            pl.BlockSpec((B, 128), lambda i: (i, 0)),
            pl.BlockSpec((B, 128), lambda i: (i, 0)),
            pl.BlockSpec((8, HID), lambda i: (0, 0)),
        ],
        out_shape=[
            jax.ShapeDtypeStruct((NP, 128), jnp.float32),
            jax.ShapeDtypeStruct((NP, 128), jnp.float32),
            jax.ShapeDtypeStruct((8, HID), jnp.float32),
        ],
    )(h2, st, gamma, beta)


def _head_body(pool_ref, wm1_ref, bm1_ref, wm2_ref, bm2_ref, out_ref):
    g = pool_ref[...] * (1.0 / N)
    a = jnp.maximum(
        jnp.dot(g, wm1_ref[...], preferred_element_type=jnp.float32) + bm1_ref[...], 0.0)
    out_ref[...] = jnp.dot(a, wm2_ref[...], preferred_element_type=jnp.float32) + bm2_ref[...]


def _head(pool, wm1, bm1, wm2p, bm2p):
    return pl.pallas_call(
        _head_body,
        grid=(1,),
        in_specs=[
            pl.BlockSpec((8, HID), lambda i: (0, 0)),
            pl.BlockSpec((HID, HID), lambda i: (0, 0)),
            pl.BlockSpec((1, HID), lambda i: (0, 0)),
            pl.BlockSpec((HID, 128), lambda i: (0, 0)),
            pl.BlockSpec((1, 128), lambda i: (0, 0)),
        ],
        out_specs=pl.BlockSpec((8, 128), lambda i: (0, 0)),
        out_shape=jax.ShapeDtypeStruct((8, 128), jnp.float32),
    )(pool, wm1, bm1, wm2p, bm2p)


def kernel(atomic_number, other_feats, edge_index, edge_attr, params):
    layers = params["layers"]
    src = edge_index[0]
    dst = edge_index[1]

    an2d = jnp.pad(atomic_number.astype(jnp.int32), (0, NP - N)).reshape(NP, 1)
    emb_pad = jnp.pad(params["emb"], ((0, NUM_AT_PAD - 100), (0, 0)))
    w0p = jnp.pad(layers[0]["We"], ((0, 0), (0, 128 - 72)))
    wcat = jnp.concatenate([w0p] + [layers[i]["We"] for i in (1, 2, 3)], axis=1)

    x00 = _embed(an2d, emb_pad)
    x01 = jnp.pad(other_feats, ((0, NP - N), (0, 64 - 8)))

    e_halves = _edge_mlp(edge_attr, wcat)

    x0, x1 = x00, x01
    d = 64
    pool = None
    for li, l in enumerate(layers):
        e0 = e_halves[2 * li]
        e1 = e_halves[2 * li + 1]
        xf = jnp.concatenate([x0, x1], axis=1)
        ef = jnp.concatenate([e0, e1], axis=1)
        m = jax.nn.relu(xf[src] + ef)
        agg = jax.ops.segment_sum(m, dst, num_segments=NP)
        a0, a1 = agg[:, :d], agg[:, d:]

        din = 72 if li == 0 else HID
        w1p = jnp.pad(l["W1"], ((0, 2 * d - din), (0, 0)))
        h2, st = _node_mlp(x0, x1, a0, a1, w1p,
                           l["b1"].reshape(1, HID), l["W2"],
                           l["b2"].reshape(1, HID), d)
        x0, x1, pool = _bn_apply(h2, st, l["gamma"].reshape(1, HID),
                                 l["beta"].reshape(1, HID))
        d = 128

    wm2p = jnp.pad(params["Wm2"], ((0, 0), (0, 127)))
    bm2p = jnp.pad(params["bm2"].reshape(1, 1), ((0, 0), (0, 127)))
    out = _head(pool, params["Wm1"], params["bm1"].reshape(1, HID), wm2p, bm2p)
    return out[0, 0].reshape(1)


# speedup vs baseline: 2.1411x; 2.1411x over previous
import functools

import jax
import jax.numpy as jnp
from jax import lax
from jax.experimental import pallas as pl
from jax.experimental.pallas import tpu as pltpu
from jax.experimental.pallas import tpu_sc as plsc

N = 10000
NP = 10240
E = 320000
HID = 256
ED = 16
NUM_AT_PAD = 104


def _embed_body(an_ref, of_ref, emb_ref, x0_ref):
    an = an_ref[...]
    ids = jax.lax.broadcasted_iota(jnp.int32, (1, NUM_AT_PAD), 1)
    oh = (an == ids).astype(jnp.float32)
    emb = jnp.dot(oh, emb_ref[...], preferred_element_type=jnp.float32)
    B = emb.shape[0]
    x0_ref[...] = jnp.concatenate(
        [emb, of_ref[...], jnp.zeros((B, 56), jnp.float32)], axis=1)


def _embed(an2d, of_pad, emb_pad):
    B = 1024
    return pl.pallas_call(
        _embed_body,
        grid=(NP // B,),
        in_specs=[
            pl.BlockSpec((B, 1), lambda i: (i, 0)),
            pl.BlockSpec((B, 8), lambda i: (i, 0)),
            pl.BlockSpec((NUM_AT_PAD, 64), lambda i: (0, 0)),
        ],
        out_specs=pl.BlockSpec((B, 128), lambda i: (i, 0)),
        out_shape=jax.ShapeDtypeStruct((NP, 128), jnp.float32),
    )(an2d, of_pad, emb_pad)


def _edge_mlp_body(ea_ref, w_ref, *out_refs):
    e = jnp.dot(ea_ref[...], w_ref[...], preferred_element_type=jnp.float32)
    for k, r in enumerate(out_refs):
        r[...] = e[:, 128 * k:128 * (k + 1)]


def _edge_mlp(edge_attr, wcat):
    B = 2000
    return pl.pallas_call(
        _edge_mlp_body,
        grid=(E // B,),
        in_specs=[
            pl.BlockSpec((B, ED), lambda i: (i, 0)),
            pl.BlockSpec((ED, 896), lambda i: (0, 0)),
        ],
        out_specs=[pl.BlockSpec((B, 128), lambda i: (i, 0)) for _ in range(7)],
        out_shape=[jax.ShapeDtypeStruct((E, 128), jnp.float32)
                   for _ in range(7)],
    )(edge_attr, wcat)


_SC_C = 80
_SC_RPT = NP // 16


def _sc_relu_add(xbuf, ebuf, D):
    @pl.loop(0, _SC_C)
    def _edge(i):
        for j in range(D // 16):
            sl = pl.ds(j * 16, 16)
            xbuf[i, sl] = jnp.maximum(xbuf[i, sl] + ebuf[i, sl], 0.0)


def _sc_msg_body(x0, x1, srcx, dstx, e0, e1, zrows, out0, out1, agg_sh,
                 idxs, idxd, xbuf, ebuf, sem):
    c = lax.axis_index("c")
    s = lax.axis_index("s")

    r0 = s * _SC_RPT
    pltpu.sync_copy(zrows, agg_sh.at[pl.ds(r0, _SC_RPT)])
    plsc.subcore_barrier()

    ept = E // 16
    ebase = s * ept

    @pl.loop(0, ept // _SC_C)
    def _chunk(k):
        off = ebase + k * _SC_C
        pltpu.sync_copy(srcx.at[pl.ds(off, _SC_C)], idxs)
        pltpu.sync_copy(dstx.at[pl.ds(off, _SC_C)], idxd)

        @pl.when(c == 0)
        def _():
            pltpu.sync_copy(e0.at[pl.ds(off, _SC_C)], ebuf)
            pltpu.async_copy(x0.at[idxs], xbuf, sem).wait()

        @pl.when(c == 1)
        def _():
            pltpu.sync_copy(e1.at[pl.ds(off, _SC_C)], ebuf)
            pltpu.async_copy(x1.at[idxs], xbuf, sem).wait()

        _sc_relu_add(xbuf, ebuf, 128)
        pltpu.sync_copy(xbuf, agg_sh.at[idxd], add=True)

    plsc.subcore_barrier()
    for j in range(_SC_RPT // _SC_C):
        sl = pl.ds(r0 + j * _SC_C, _SC_C)

        @pl.when(c == 0)
        def _():
            pltpu.sync_copy(agg_sh.at[sl], out0.at[sl])

        @pl.when(c == 1)
        def _():
            pltpu.sync_copy(agg_sh.at[sl], out1.at[sl])


def _sc_msg0_body(x0, srcx, dstx, e0, zrows, out0, out1, agg_sh,
                  idxs, idxd, xbuf, ebuf, sem):
    c = lax.axis_index("c")
    s = lax.axis_index("s")

    r0 = s * _SC_RPT
    pltpu.sync_copy(zrows, agg_sh.at[pl.ds(r0, _SC_RPT)])
    plsc.subcore_barrier()

    ept = E // 32
    ebase = (c * 16 + s) * ept

    @pl.loop(0, ept // _SC_C)
    def _chunk(k):
        off = ebase + k * _SC_C
        pltpu.sync_copy(srcx.at[pl.ds(off, _SC_C)], idxs)
        pltpu.sync_copy(dstx.at[pl.ds(off, _SC_C)], idxd)
        pltpu.sync_copy(e0.at[pl.ds(off, _SC_C)], ebuf)
        pltpu.async_copy(x0.at[idxs], xbuf, sem).wait()
        _sc_relu_add(xbuf, ebuf, 128)
        pltpu.sync_copy(xbuf, agg_sh.at[idxd], add=True)

    plsc.subcore_barrier()
    for j in range(_SC_RPT // _SC_C):
        sl = pl.ds(r0 + j * _SC_C, _SC_C)

        @pl.when(c == 0)
        def _():
            pltpu.sync_copy(agg_sh.at[sl], out0.at[sl])

        @pl.when(c == 1)
        def _():
            pltpu.sync_copy(agg_sh.at[sl], out1.at[sl])


def _sc_mesh():
    return plsc.VectorSubcoreMesh(core_axis_name="c", subcore_axis_name="s",
                                  num_cores=2, num_subcores=16)


def _sc_scratch():
    return [
        pltpu.VMEM_SHARED((NP, 128), jnp.float32),
        pltpu.VMEM((_SC_C,), jnp.int32),
        pltpu.VMEM((_SC_C,), jnp.int32),
        pltpu.VMEM((_SC_C, 128), jnp.float32),
        pltpu.VMEM((_SC_C, 128), jnp.float32),
        pltpu.SemaphoreType.DMA,
    ]


@functools.cache
def _sc_msg():
    return pl.kernel(
        _sc_msg_body,
        out_type=[jax.ShapeDtypeStruct((NP, 128), jnp.float32),
                  jax.ShapeDtypeStruct((NP, 128), jnp.float32)],
        mesh=_sc_mesh(),
        scratch_types=_sc_scratch(),
    )


@functools.cache
def _sc_msg0():
    return pl.kernel(
        _sc_msg0_body,
        out_type=[jax.ShapeDtypeStruct((NP, 128), jnp.float32),
                  jax.ShapeDtypeStruct((NP, 128), jnp.float32)],
        mesh=_sc_mesh(),
        scratch_types=_sc_scratch(),
    )


def _mlp_body(x0_ref, x1_ref, a0_ref, a1_ref, w1_ref, b1_ref, w2_ref, b2_ref,
              h2_ref, st_ref, *, B, layer0):
    if layer0:
        h = x0_ref[...] + a0_ref[...] + a1_ref[...]
    else:
        h = jnp.concatenate(
            [x0_ref[...] + a0_ref[...], x1_ref[...] + a1_ref[...]], axis=1)
    h1 = jnp.maximum(
        jnp.dot(h, w1_ref[...], preferred_element_type=jnp.float32) + b1_ref[...], 0.0)
    h2 = jnp.dot(h1, w2_ref[...], preferred_element_type=jnp.float32) + b2_ref[...]
    h2_ref[...] = h2
    gid = pl.program_id(0)
    rows = gid * B + jax.lax.broadcasted_iota(jnp.int32, (B, 1), 0)
    hm = h2 * (rows < N).astype(jnp.float32)

    @pl.when(gid == 0)
    def _():
        st_ref[...] = jnp.zeros_like(st_ref)

    st_ref[0:1, :] += jnp.sum(hm, axis=0, keepdims=True)
    st_ref[1:2, :] += jnp.sum(hm * hm, axis=0, keepdims=True)


def _node_mlp(x0, x1, a0, a1, w1p, b1, w2, b2, layer0):
    B = 512
    din = 128 if layer0 else 256
    return pl.pallas_call(
        functools.partial(_mlp_body, B=B, layer0=layer0),
        grid=(NP // B,),
        in_specs=[
            pl.BlockSpec((B, 128), lambda i: (i, 0)),
            pl.BlockSpec((B, 128), lambda i: (i, 0)),
            pl.BlockSpec((B, 128), lambda i: (i, 0)),
            pl.BlockSpec((B, 128), lambda i: (i, 0)),
            pl.BlockSpec((din, HID), lambda i: (0, 0)),
            pl.BlockSpec((1, HID), lambda i: (0, 0)),
            pl.BlockSpec((HID, HID), lambda i: (0, 0)),
            pl.BlockSpec((1, HID), lambda i: (0, 0)),
        ],
        out_specs=[
            pl.BlockSpec((B, HID), lambda i: (i, 0)),
            pl.BlockSpec((8, HID), lambda i: (0, 0)),
        ],
        out_shape=[
            jax.ShapeDtypeStruct((NP, HID), jnp.float32),
            jax.ShapeDtypeStruct((8, HID), jnp.float32),
        ],
    )(x0, x1, a0, a1, w1p, b1, w2, b2)


def _bn_body(h2_ref, st_ref, g_ref, bt_ref, y0_ref, y1_ref, pool_ref, *, B):
    inv_n = 1.0 / N
    mean = st_ref[0:1, :] * inv_n
    var = st_ref[1:2, :] * inv_n - mean * mean
    inv = jax.lax.rsqrt(var + 1e-5)
    y = jnp.maximum((h2_ref[...] - mean) * inv * g_ref[...] + bt_ref[...], 0.0)
    y0_ref[...] = y[:, :128]
    y1_ref[...] = y[:, 128:]
    gid = pl.program_id(0)
    rows = gid * B + jax.lax.broadcasted_iota(jnp.int32, (B, 1), 0)
    ym = y * (rows < N).astype(jnp.float32)

    @pl.when(gid == 0)
    def _():
        pool_ref[...] = jnp.zeros_like(pool_ref)

    pool_ref[0:1, :] += jnp.sum(ym, axis=0, keepdims=True)


def _bn_apply(h2, st, gamma, beta):
    B = 512
    return pl.pallas_call(
        functools.partial(_bn_body, B=B),
        grid=(NP // B,),
        in_specs=[
            pl.BlockSpec((B, HID), lambda i: (i, 0)),
            pl.BlockSpec((8, HID), lambda i: (0, 0)),
            pl.BlockSpec((1, HID), lambda i: (0, 0)),
            pl.BlockSpec((1, HID), lambda i: (0, 0)),
        ],
        out_specs=[
            pl.BlockSpec((B, 128), lambda i: (i, 0)),
            pl.BlockSpec((B, 128), lambda i: (i, 0)),
            pl.BlockSpec((8, HID), lambda i: (0, 0)),
        ],
        out_shape=[
            jax.ShapeDtypeStruct((NP, 128), jnp.float32),
            jax.ShapeDtypeStruct((NP, 128), jnp.float32),
            jax.ShapeDtypeStruct((8, HID), jnp.float32),
        ],
    )(h2, st, gamma, beta)


def _head_body(pool_ref, wm1_ref, bm1_ref, wm2_ref, bm2_ref, out_ref):
    g = pool_ref[...] * (1.0 / N)
    a = jnp.maximum(
        jnp.dot(g, wm1_ref[...], preferred_element_type=jnp.float32) + bm1_ref[...], 0.0)
    out_ref[...] = jnp.dot(a, wm2_ref[...], preferred_element_type=jnp.float32) + bm2_ref[...]


def _head(pool, wm1, bm1, wm2p, bm2p):
    return pl.pallas_call(
        _head_body,
        grid=(1,),
        in_specs=[
            pl.BlockSpec((8, HID), lambda i: (0, 0)),
            pl.BlockSpec((HID, HID), lambda i: (0, 0)),
            pl.BlockSpec((1, HID), lambda i: (0, 0)),
            pl.BlockSpec((HID, 128), lambda i: (0, 0)),
            pl.BlockSpec((1, 128), lambda i: (0, 0)),
        ],
        out_specs=pl.BlockSpec((8, 128), lambda i: (0, 0)),
        out_shape=jax.ShapeDtypeStruct((8, 128), jnp.float32),
    )(pool, wm1, bm1, wm2p, bm2p)


def kernel(atomic_number, other_feats, edge_index, edge_attr, params):
    layers = params["layers"]
    src = edge_index[0].astype(jnp.int32)
    dst = edge_index[1].astype(jnp.int32)

    an2d = jnp.pad(atomic_number.astype(jnp.int32), (0, NP - N)).reshape(NP, 1)
    of_pad = jnp.pad(other_feats, ((0, NP - N), (0, 0)))
    emb_pad = jnp.pad(params["emb"], ((0, NUM_AT_PAD - 100), (0, 0)))
    w0p = jnp.pad(layers[0]["We"], ((0, 0), (0, 128 - 72)))
    wcat = jnp.concatenate([w0p] + [layers[i]["We"] for i in (1, 2, 3)], axis=1)

    x0 = _embed(an2d, of_pad, emb_pad)
    x1 = None
    e_parts = _edge_mlp(edge_attr, wcat)

    zrows = jnp.zeros((_SC_RPT, 128), jnp.float32)
    pool = None
    for li, l in enumerate(layers):
        if li == 0:
            a0, a1 = _sc_msg0()(x0, src, dst, e_parts[0], zrows)
        else:
            a0, a1 = _sc_msg()(x0, x1, src, dst,
                               e_parts[2 * li - 1], e_parts[2 * li], zrows)

        din = 72 if li == 0 else HID
        din_pad = 128 if li == 0 else HID
        w1p = jnp.pad(l["W1"], ((0, din_pad - din), (0, 0)))
        h2, st = _node_mlp(x0, x0 if x1 is None else x1, a0, a1, w1p,
                           l["b1"].reshape(1, HID), l["W2"],
                           l["b2"].reshape(1, HID), li == 0)
        x0, x1, pool = _bn_apply(h2, st, l["gamma"].reshape(1, HID),
                                 l["beta"].reshape(1, HID))

    wm2p = jnp.pad(params["Wm2"], ((0, 0), (0, 127)))
    bm2p = jnp.pad(params["bm2"].reshape(1, 1), ((0, 0), (0, 127)))
    out = _head(pool, params["Wm1"], params["bm1"].reshape(1, HID), wm2p, bm2p)
    return out[0, 0].reshape(1)
